# baseline (speedup 1.0000x reference)
import functools

import jax
import jax.numpy as jnp
from jax import lax
from jax.experimental import pallas as pl
from jax.experimental.pallas import tpu as pltpu
from jax.experimental.pallas import tpu_sc as plsc

N = 10000
E = 320000
D_IN = 128
D_H = 64
D_OUT = 32

NC = 2
NS = 16
W = 128
KW = 80
EP = NC * NS * KW * W
NP = 10240
RPT = NP // NS

_MESH = plsc.VectorSubcoreMesh(core_axis_name="c", subcore_axis_name="s")
_SC_PARAMS = pltpu.CompilerParams(use_tc_tiling_on_sc=False)


def _sc_degree(dstw, ones1, zeros1):
    @functools.partial(
        pl.kernel,
        out_type=jax.ShapeDtypeStruct((NP, 128), jnp.float32),
        mesh=_MESH,
        compiler_params=_SC_PARAMS,
        scratch_types=[
            pltpu.VMEM((KW, W), jnp.int32),
            pltpu.VMEM((W, 16), jnp.float32),
            pltpu.VMEM_SHARED((NP, 16), jnp.float32),
        ],
    )
    def k(dst_hbm, ones_hbm, z_hbm, out_hbm, dst_v, ones_v, acc):
        cid = lax.axis_index("c")
        sid = lax.axis_index("s")
        base = sid * RPT
        pltpu.sync_copy(z_hbm.at[pl.ds(base, RPT)], acc.at[pl.ds(base, RPT)])
        pltpu.sync_copy(ones_hbm, ones_v)
        pltpu.sync_copy(dst_hbm.at[cid].at[sid], dst_v)
        plsc.subcore_barrier()

        @pl.loop(0, KW)
        def _(t):
            pltpu.sync_copy(ones_v, acc.at[dst_v.at[t]], add=True)

        plsc.subcore_barrier()
        pltpu.sync_copy(acc.at[pl.ds(base, RPT)],
                        out_hbm.at[pl.ds(base, RPT), pl.ds(cid * 16, 16)])

    return k(dstw, ones1, zeros1)


KWF = EP // (NS * W)


def _sc_segment_out(y2, srcw2, dstw2, zeros32, d16, b2, flag):
    @functools.partial(
        pl.kernel,
        out_type=jax.ShapeDtypeStruct((NP, 128), jnp.float32),
        mesh=_MESH,
        compiler_params=_SC_PARAMS,
        scratch_types=[
            pltpu.VMEM((KWF, W), jnp.int32),
            pltpu.VMEM((KWF, W), jnp.int32),
            pltpu.VMEM((W, D_OUT), jnp.float32),
            pltpu.VMEM((W, D_OUT), jnp.float32),
            pltpu.VMEM((W, D_OUT), jnp.float32),
            pltpu.VMEM((W, D_OUT), jnp.float32),
            pltpu.VMEM((W, 16), jnp.float32),
            pltpu.VMEM((D_OUT,), jnp.float32),
            pltpu.VMEM((16,), jnp.float32),
            pltpu.VMEM_SHARED((NP, D_OUT), jnp.float32),
            pltpu.VMEM_SHARED((NP, D_OUT), jnp.float32),
            pltpu.SemaphoreType.DMA,
            pltpu.SemaphoreType.DMA,
        ],
    )
    def k(y_hbm, src_hbm, dst_hbm, z_hbm, d16_hbm, b_hbm, f_hbm, out_hbm,
          src_v, dst_v, rows0, rows1, acc_v, y_v, d_v, b_v, f_v,
          acc, y_sh, sem0, sem1):
        cid = lax.axis_index("c")
        sid = lax.axis_index("s")
        base = sid * RPT
        ybase = sid * (N // NS)
        pltpu.sync_copy(y_hbm.at[cid].at[pl.ds(ybase, N // NS)],
                        y_sh.at[pl.ds(ybase, N // NS)])
        pltpu.sync_copy(z_hbm.at[pl.ds(base, RPT)], acc.at[pl.ds(base, RPT)])
        pltpu.sync_copy(src_hbm.at[sid], src_v)
        pltpu.sync_copy(dst_hbm.at[sid], dst_v)
        pltpu.sync_copy(b_hbm.at[cid], b_v)
        pltpu.sync_copy(f_hbm, f_v)
        plsc.subcore_barrier()

        pltpu.async_copy(y_sh.at[src_v.at[0]], rows0, sem0)

        @pl.loop(0, KWF, step=2)
        def _(t):
            pltpu.make_async_copy(y_sh.at[src_v.at[t]], rows0, sem0).wait()
            pltpu.async_copy(y_sh.at[src_v.at[t + 1]], rows1, sem1)
            pltpu.sync_copy(rows0, acc.at[dst_v.at[t]], add=True)
            pltpu.make_async_copy(y_sh.at[src_v.at[t + 1]], rows1, sem1).wait()

            @pl.when(t + 2 < KWF)
            def _():
                pltpu.async_copy(y_sh.at[src_v.at[t + 2]], rows0, sem0)

            pltpu.sync_copy(rows1, acc.at[dst_v.at[t + 1]], add=True)

        plsc.subcore_barrier()

        bv0 = b_v[pl.ds(0, 16)]
        bv1 = b_v[pl.ds(16, 16)]
        fv = f_v[...]

        @pl.loop(0, RPT, step=W)
        def _(rb):
            pltpu.sync_copy(acc.at[pl.ds(base + rb, W)], acc_v)
            pltpu.sync_copy(y_sh.at[pl.ds(base + rb, W)], y_v)
            pltpu.sync_copy(d16_hbm.at[pl.ds(base + rb, W)], d_v)

            @pl.loop(0, W)
            def _(r):
                dv = d_v[r]
                a0 = acc_v[r, pl.ds(0, 16)]
                y0 = y_v[r, pl.ds(0, 16)]
                o0 = dv * (a0 + y0) + bv0
                a1 = acc_v[r, pl.ds(16, 16)]
                y1 = y_v[r, pl.ds(16, 16)]
                o1 = dv * (a1 + y1) + bv1
                acc_v[r, pl.ds(0, 16)] = jnp.maximum(o0, fv * o0)
                acc_v[r, pl.ds(16, 16)] = jnp.maximum(o1, fv * o1)

            pltpu.sync_copy(
                acc_v,
                out_hbm.at[pl.ds(base + rb, W), pl.ds(cid * D_OUT, D_OUT)])

    return k(y2, srcw2, dstw2, zeros32, d16, b2, flag)


def _tc_layer2(h128, d, Wcat):
    def body(h_ref, d_ref, w_ref, y2_ref):
        h = h_ref[:N, :D_H]
        y2 = jax.lax.dot_general(
            h, w_ref[...], (((1,), (0,)), ((), ())),
            preferred_element_type=jnp.float32,
            precision=lax.Precision.HIGHEST) * d_ref[...]
        y2_ref[0] = y2[:, :D_OUT]
        y2_ref[1] = y2[:, D_OUT:]

    return pl.pallas_call(
        body,
        out_shape=jax.ShapeDtypeStruct((2, N, D_OUT), jnp.float32),
    )(h128, d, Wcat)


def _tc_layer1(cnt, x, W1):
    def body(c_ref, x_ref, w_ref, y_ref, d_ref, d16_ref):
        c = c_ref[:, 0:1] + c_ref[:, 16:17]
        d_full = lax.rsqrt(c + 1.0)
        d = d_full[:N]
        u = jax.lax.dot_general(
            x_ref[...], w_ref[...], (((1,), (0,)), ((), ())),
            preferred_element_type=jnp.float32,
            precision=lax.Precision.HIGHEST)
        y1 = u * d
        y_ref[0] = y1[:, :D_H // 2]
        y_ref[1] = y1[:, D_H // 2:]
        d_ref[...] = d
        d16_ref[...] = jnp.broadcast_to(d_full, (NP, 16))

    return pl.pallas_call(
        body,
        out_shape=(jax.ShapeDtypeStruct((2, N, D_H // 2), jnp.float32),
                   jax.ShapeDtypeStruct((N, 1), jnp.float32),
                   jax.ShapeDtypeStruct((NP, 16), jnp.float32)),
    )(cnt, x, W1)


def kernel(x, edge_index, W1, b1, Wmu, bmu, Wls, bls):
    src = edge_index[0]
    dst = edge_index[1]
    pad = EP - E
    pad_dst = N + (jnp.arange(pad, dtype=jnp.int32) % (NP - N))
    src_p = jnp.concatenate([src, jnp.zeros((pad,), jnp.int32)])
    dst_p = jnp.concatenate([dst, pad_dst])
    dstw = dst_p.reshape(NC, NS, KW, W)
    srcw2 = src_p.reshape(NS, KWF, W)
    dstw2 = dst_p.reshape(NS, KWF, W)
    zeros1 = jnp.zeros((NP, 16), jnp.float32)
    zeros32 = jnp.zeros((NP, D_OUT), jnp.float32)
    ones1 = jnp.zeros((W, 16), jnp.float32).at[:, 0].set(1.0)

    cnt = _sc_degree(dstw, ones1, zeros1)
    y1, d, d16 = _tc_layer1(cnt, x, W1)
    h128 = _sc_segment_out(y1, srcw2, dstw2, zeros32, d16,
                           b1.reshape(2, D_H // 2),
                           jnp.zeros((16,), jnp.float32))
    Wcat = jnp.concatenate([Wmu, Wls], axis=1)
    y2 = _tc_layer2(h128, d, Wcat)
    out = _sc_segment_out(y2, srcw2, dstw2, zeros32, d16,
                          jnp.stack([bmu, bls]),
                          jnp.ones((16,), jnp.float32))
    return (out[:N, :D_OUT], out[:N, D_OUT:2 * D_OUT])


# speedup vs baseline: 1.1098x; 1.1098x over previous
import functools

import jax
import jax.numpy as jnp
from jax import lax
from jax.experimental import pallas as pl
from jax.experimental.pallas import tpu as pltpu
from jax.experimental.pallas import tpu_sc as plsc

N = 10000
E = 320000
D_IN = 128
D_H = 64
D_OUT = 32

NC = 2
NS = 16
W = 128
KW = 80
EP = NC * NS * KW * W
NP = 10240
RPT = NP // NS

_MESH = plsc.VectorSubcoreMesh(core_axis_name="c", subcore_axis_name="s")
_SC_PARAMS = pltpu.CompilerParams(use_tc_tiling_on_sc=False)


def _sc_degree(dstw, ones1, zeros1):
    @functools.partial(
        pl.kernel,
        out_type=jax.ShapeDtypeStruct((NP, 128), jnp.float32),
        mesh=_MESH,
        compiler_params=_SC_PARAMS,
        scratch_types=[
            pltpu.VMEM((KW, W), jnp.int32),
            pltpu.VMEM((W, 16), jnp.float32),
            pltpu.VMEM_SHARED((NP, 16), jnp.float32),
        ],
    )
    def k(dst_hbm, ones_hbm, z_hbm, out_hbm, dst_v, ones_v, acc):
        cid = lax.axis_index("c")
        sid = lax.axis_index("s")
        base = sid * RPT
        pltpu.sync_copy(z_hbm.at[pl.ds(base, RPT)], acc.at[pl.ds(base, RPT)])
        pltpu.sync_copy(ones_hbm, ones_v)
        pltpu.sync_copy(dst_hbm.at[cid].at[sid], dst_v)
        plsc.subcore_barrier()

        @pl.loop(0, KW)
        def _(t):
            pltpu.sync_copy(ones_v, acc.at[dst_v.at[t]], add=True)

        plsc.subcore_barrier()
        pltpu.sync_copy(acc.at[pl.ds(base, RPT)],
                        out_hbm.at[pl.ds(base, RPT), pl.ds(cid * 16, 16)])

    return k(dstw, ones1, zeros1)


KWF = EP // (NS * W)


def _sc_segment_sum(a1, srcw, dstw, zeros64):
    @functools.partial(
        pl.kernel,
        out_type=jax.ShapeDtypeStruct((NP, 128), jnp.float32),
        mesh=_MESH,
        compiler_params=_SC_PARAMS,
        scratch_types=[
            pltpu.VMEM((KW, W), jnp.int32),
            pltpu.VMEM((KW, W), jnp.int32),
            pltpu.VMEM((W, D_H), jnp.float32),
            pltpu.VMEM((W, D_H), jnp.float32),
            pltpu.VMEM_SHARED((NP, D_H), jnp.float32),
            pltpu.VMEM_SHARED((N, D_H), jnp.float32),
            pltpu.SemaphoreType.DMA,
            pltpu.SemaphoreType.DMA,
        ],
    )
    def k(a_hbm, src_hbm, dst_hbm, z_hbm, out_hbm,
          src_v, dst_v, rows0, rows1, acc, y_sh, sem0, sem1):
        cid = lax.axis_index("c")
        sid = lax.axis_index("s")
        base = sid * RPT
        ybase = sid * (N // NS)
        pltpu.sync_copy(a_hbm.at[pl.ds(ybase, N // NS), pl.ds(0, D_H)],
                        y_sh.at[pl.ds(ybase, N // NS)])
        pltpu.sync_copy(z_hbm.at[pl.ds(base, RPT)], acc.at[pl.ds(base, RPT)])
        pltpu.sync_copy(src_hbm.at[cid].at[sid], src_v)
        pltpu.sync_copy(dst_hbm.at[cid].at[sid], dst_v)
        plsc.subcore_barrier()

        pltpu.async_copy(y_sh.at[src_v.at[0]], rows0, sem0)

        @pl.loop(0, KW, step=2)
        def _(t):
            pltpu.make_async_copy(y_sh.at[src_v.at[t]], rows0, sem0).wait()
            pltpu.async_copy(y_sh.at[src_v.at[t + 1]], rows1, sem1)
            pltpu.sync_copy(rows0, acc.at[dst_v.at[t]], add=True)
            pltpu.make_async_copy(y_sh.at[src_v.at[t + 1]], rows1, sem1).wait()

            @pl.when(t + 2 < KW)
            def _():
                pltpu.async_copy(y_sh.at[src_v.at[t + 2]], rows0, sem0)

            pltpu.sync_copy(rows1, acc.at[dst_v.at[t + 1]], add=True)

        plsc.subcore_barrier()
        pltpu.sync_copy(acc.at[pl.ds(base, RPT)],
                        out_hbm.at[pl.ds(base, RPT), pl.ds(cid * D_H, D_H)])

    return k(a1, srcw, dstw, zeros64)


def _sc_segment_out(y2, srcw2, dstw2, zeros32, d16, b2):
    @functools.partial(
        pl.kernel,
        out_type=jax.ShapeDtypeStruct((NP, 128), jnp.float32),
        mesh=_MESH,
        compiler_params=_SC_PARAMS,
        scratch_types=[
            pltpu.VMEM((KWF, W), jnp.int32),
            pltpu.VMEM((KWF, W), jnp.int32),
            pltpu.VMEM((W, D_OUT), jnp.float32),
            pltpu.VMEM((W, D_OUT), jnp.float32),
            pltpu.VMEM((W, D_OUT), jnp.float32),
            pltpu.VMEM((W, D_OUT), jnp.float32),
            pltpu.VMEM((W, 16), jnp.float32),
            pltpu.VMEM((D_OUT,), jnp.float32),
            pltpu.VMEM_SHARED((NP, D_OUT), jnp.float32),
            pltpu.VMEM_SHARED((NP, D_OUT), jnp.float32),
            pltpu.SemaphoreType.DMA,
            pltpu.SemaphoreType.DMA,
        ],
    )
    def k(y_hbm, src_hbm, dst_hbm, z_hbm, d16_hbm, b_hbm, out_hbm,
          src_v, dst_v, rows0, rows1, acc_v, y_v, d_v, b_v,
          acc, y_sh, sem0, sem1):
        cid = lax.axis_index("c")
        sid = lax.axis_index("s")
        base = sid * RPT
        ybase = sid * (N // NS)
        pltpu.sync_copy(
            y_hbm.at[pl.ds(ybase, N // NS), pl.ds(cid * D_OUT, D_OUT)],
            y_sh.at[pl.ds(ybase, N // NS)])
        pltpu.sync_copy(z_hbm.at[pl.ds(base, RPT)], acc.at[pl.ds(base, RPT)])
        pltpu.sync_copy(src_hbm.at[sid], src_v)
        pltpu.sync_copy(dst_hbm.at[sid], dst_v)
        pltpu.sync_copy(b_hbm.at[cid], b_v)
        plsc.subcore_barrier()

        pltpu.async_copy(y_sh.at[src_v.at[0]], rows0, sem0)

        @pl.loop(0, KWF, step=2)
        def _(t):
            pltpu.make_async_copy(y_sh.at[src_v.at[t]], rows0, sem0).wait()
            pltpu.async_copy(y_sh.at[src_v.at[t + 1]], rows1, sem1)
            pltpu.sync_copy(rows0, acc.at[dst_v.at[t]], add=True)
            pltpu.make_async_copy(y_sh.at[src_v.at[t + 1]], rows1, sem1).wait()

            @pl.when(t + 2 < KWF)
            def _():
                pltpu.async_copy(y_sh.at[src_v.at[t + 2]], rows0, sem0)

            pltpu.sync_copy(rows1, acc.at[dst_v.at[t + 1]], add=True)

        plsc.subcore_barrier()

        bv0 = b_v[pl.ds(0, 16)]
        bv1 = b_v[pl.ds(16, 16)]

        @pl.loop(0, RPT, step=W)
        def _(rb):
            pltpu.sync_copy(acc.at[pl.ds(base + rb, W)], acc_v)
            pltpu.sync_copy(y_sh.at[pl.ds(base + rb, W)], y_v)
            pltpu.sync_copy(
                d16_hbm.at[pl.ds(base + rb, W), pl.ds(D_H, 16)], d_v)

            @pl.loop(0, W)
            def _(r):
                dv = d_v[r]
                a0 = acc_v[r, pl.ds(0, 16)]
                y0 = y_v[r, pl.ds(0, 16)]
                acc_v[r, pl.ds(0, 16)] = dv * (a0 + y0) + bv0
                a1 = acc_v[r, pl.ds(16, 16)]
                y1 = y_v[r, pl.ds(16, 16)]
                acc_v[r, pl.ds(16, 16)] = dv * (a1 + y1) + bv1

            pltpu.sync_copy(
                acc_v,
                out_hbm.at[pl.ds(base + rb, W), pl.ds(cid * D_OUT, D_OUT)])

    return k(y2, srcw2, dstw2, zeros32, d16, b2)


def _tc_layer1(cnt, x, W1):
    def body(c_ref, x_ref, w_ref, a_ref):
        c = c_ref[:, 0:1] + c_ref[:, 16:17]
        d_full = lax.rsqrt(c + 1.0)
        u = jax.lax.dot_general(
            x_ref[...], w_ref[...], (((1,), (0,)), ((), ())),
            preferred_element_type=jnp.float32,
            precision=lax.Precision.HIGHEST)
        y1 = u * d_full[:N]
        a_ref[:N, :D_H] = y1
        a_ref[N:, :D_H] = jnp.zeros((NP - N, D_H), jnp.float32)
        a_ref[:, D_H:D_H + 16] = jnp.broadcast_to(d_full, (NP, 16))
        a_ref[:, D_H + 16:] = jnp.zeros((NP, 48), jnp.float32)

    return pl.pallas_call(
        body,
        out_shape=jax.ShapeDtypeStruct((NP, 128), jnp.float32),
    )(cnt, x, W1)


def _tc_layer2(parts1, a1, Wcat, b1):
    def body(p_ref, a_ref, w_ref, b_ref, o_ref):
        s1 = p_ref[:N, :D_H] + p_ref[:N, D_H:]
        d = a_ref[:N, D_H:D_H + 1]
        y1 = a_ref[:N, :D_H]
        h = jnp.maximum(d * (s1 + y1) + b_ref[...], 0.0)
        y2 = jax.lax.dot_general(
            h, w_ref[...], (((1,), (0,)), ((), ())),
            preferred_element_type=jnp.float32,
            precision=lax.Precision.HIGHEST) * d
        o_ref[:N, :D_H] = y2
        o_ref[N:, :D_H] = jnp.zeros((NP - N, D_H), jnp.float32)
        o_ref[:, D_H:] = jnp.zeros((NP, D_H), jnp.float32)

    return pl.pallas_call(
        body,
        out_shape=jax.ShapeDtypeStruct((NP, 128), jnp.float32),
    )(parts1, a1, Wcat, b1)


def kernel(x, edge_index, W1, b1, Wmu, bmu, Wls, bls):
    src = edge_index[0]
    dst = edge_index[1]
    pad = EP - E
    pad_dst = N + (jnp.arange(pad, dtype=jnp.int32) % (NP - N))
    src_p = jnp.concatenate([src, jnp.zeros((pad,), jnp.int32)])
    dst_p = jnp.concatenate([dst, pad_dst])
    srcw = src_p.reshape(NC, NS, KW, W)
    dstw = dst_p.reshape(NC, NS, KW, W)
    srcw2 = src_p.reshape(NS, KWF, W)
    dstw2 = dst_p.reshape(NS, KWF, W)
    zeros1 = jnp.zeros((NP, 16), jnp.float32)
    zeros64 = jnp.zeros((NP, D_H), jnp.float32)
    zeros32 = jnp.zeros((NP, D_OUT), jnp.float32)
    ones1 = jnp.zeros((W, 16), jnp.float32).at[:, 0].set(1.0)

    cnt = _sc_degree(dstw, ones1, zeros1)
    a1 = _tc_layer1(cnt, x, W1)
    parts1 = _sc_segment_sum(a1, srcw, dstw, zeros64)
    Wcat = jnp.concatenate([Wmu, Wls], axis=1)
    y2 = _tc_layer2(parts1, a1, Wcat, b1.reshape(1, D_H))
    out = _sc_segment_out(y2, srcw2, dstw2, zeros32, a1,
                          jnp.stack([bmu, bls]))
    return (out[:N, :D_OUT], out[:N, D_OUT:2 * D_OUT])


# speedup vs baseline: 1.1183x; 1.0077x over previous
import functools

import jax
import jax.numpy as jnp
from jax import lax
from jax.experimental import pallas as pl
from jax.experimental.pallas import tpu as pltpu
from jax.experimental.pallas import tpu_sc as plsc

N = 10000
E = 320000
D_IN = 128
D_H = 64
D_OUT = 32

NC = 2
NS = 16
W = 128
KW = 80
EP = NC * NS * KW * W
NP = 10240
RPT = NP // NS

_MESH = plsc.VectorSubcoreMesh(core_axis_name="c", subcore_axis_name="s")
_SC_PARAMS = pltpu.CompilerParams(use_tc_tiling_on_sc=False)


def _sc_degree(dstw, ones1, zeros1):
    @functools.partial(
        pl.kernel,
        out_type=jax.ShapeDtypeStruct((NP, 128), jnp.float32),
        mesh=_MESH,
        compiler_params=_SC_PARAMS,
        scratch_types=[
            pltpu.VMEM((KW, W), jnp.int32),
            pltpu.VMEM((W, 16), jnp.float32),
            pltpu.VMEM_SHARED((NP, 16), jnp.float32),
        ],
    )
    def k(dst_hbm, ones_hbm, z_hbm, out_hbm, dst_v, ones_v, acc):
        cid = lax.axis_index("c")
        sid = lax.axis_index("s")
        base = sid * RPT
        pltpu.sync_copy(z_hbm.at[pl.ds(base, RPT)], acc.at[pl.ds(base, RPT)])
        pltpu.sync_copy(ones_hbm, ones_v)
        pltpu.sync_copy(dst_hbm.at[cid].at[sid], dst_v)
        plsc.subcore_barrier()

        @pl.loop(0, KW)
        def _(t):
            pltpu.sync_copy(ones_v, acc.at[dst_v.at[t]], add=True)

        plsc.subcore_barrier()
        pltpu.sync_copy(acc.at[pl.ds(base, RPT)],
                        out_hbm.at[pl.ds(base, RPT), pl.ds(cid * 16, 16)])

    return k(dstw, ones1, zeros1)


KWF = EP // (NS * W)


def _sc_segment_sum(a1, srcw, dstw, zeros64):
    @functools.partial(
        pl.kernel,
        out_type=jax.ShapeDtypeStruct((NP, 128), jnp.float32),
        mesh=_MESH,
        compiler_params=_SC_PARAMS,
        scratch_types=[
            pltpu.VMEM((KW, W), jnp.int32),
            pltpu.VMEM((KW, W), jnp.int32),
            pltpu.VMEM((W, D_H), jnp.float32),
            pltpu.VMEM((W, D_H), jnp.float32),
            pltpu.VMEM_SHARED((NP, D_H), jnp.float32),
            pltpu.VMEM_SHARED((N, D_H), jnp.float32),
            pltpu.SemaphoreType.DMA,
            pltpu.SemaphoreType.DMA,
        ],
    )
    def k(a_hbm, src_hbm, dst_hbm, z_hbm, out_hbm,
          src_v, dst_v, rows0, rows1, acc, y_sh, sem0, sem1):
        cid = lax.axis_index("c")
        sid = lax.axis_index("s")
        base = sid * RPT
        ybase = sid * (N // NS)
        pltpu.sync_copy(a_hbm.at[pl.ds(ybase, N // NS), pl.ds(0, D_H)],
                        y_sh.at[pl.ds(ybase, N // NS)])
        pltpu.sync_copy(z_hbm.at[pl.ds(base, RPT)], acc.at[pl.ds(base, RPT)])
        pltpu.sync_copy(src_hbm.at[cid].at[sid], src_v)
        pltpu.sync_copy(dst_hbm.at[cid].at[sid], dst_v)
        plsc.subcore_barrier()

        pltpu.async_copy(y_sh.at[src_v.at[0]], rows0, sem0)

        @pl.loop(0, KW, step=2)
        def _(t):
            pltpu.make_async_copy(y_sh.at[src_v.at[t]], rows0, sem0).wait()
            pltpu.async_copy(y_sh.at[src_v.at[t + 1]], rows1, sem1)
            pltpu.sync_copy(rows0, acc.at[dst_v.at[t]], add=True)
            pltpu.make_async_copy(y_sh.at[src_v.at[t + 1]], rows1, sem1).wait()

            @pl.when(t + 2 < KW)
            def _():
                pltpu.async_copy(y_sh.at[src_v.at[t + 2]], rows0, sem0)

            pltpu.sync_copy(rows1, acc.at[dst_v.at[t + 1]], add=True)

        plsc.subcore_barrier()
        pltpu.sync_copy(acc.at[pl.ds(base, RPT)],
                        out_hbm.at[pl.ds(base, RPT), pl.ds(cid * D_H, D_H)])

    return k(a1, srcw, dstw, zeros64)


def _sc_segment_out(y2, srcw2, dstw2, zeros32, d16, b2):
    @functools.partial(
        pl.kernel,
        out_type=(jax.ShapeDtypeStruct((N, D_OUT), jnp.float32),
                  jax.ShapeDtypeStruct((N, D_OUT), jnp.float32)),
        mesh=_MESH,
        compiler_params=_SC_PARAMS,
        scratch_types=[
            pltpu.VMEM((KWF, W), jnp.int32),
            pltpu.VMEM((KWF, W), jnp.int32),
            pltpu.VMEM((W, D_OUT), jnp.float32),
            pltpu.VMEM((W, D_OUT), jnp.float32),
            pltpu.VMEM((W, D_OUT), jnp.float32),
            pltpu.VMEM((W, D_OUT), jnp.float32),
            pltpu.VMEM((W, 16), jnp.float32),
            pltpu.VMEM((D_OUT,), jnp.float32),
            pltpu.VMEM_SHARED((NP, D_OUT), jnp.float32),
            pltpu.VMEM_SHARED((NP, D_OUT), jnp.float32),
            pltpu.SemaphoreType.DMA,
            pltpu.SemaphoreType.DMA,
        ],
    )
    def k(y_hbm, src_hbm, dst_hbm, z_hbm, d16_hbm, b_hbm, mu_hbm, ls_hbm,
          src_v, dst_v, rows0, rows1, acc_v, y_v, d_v, b_v,
          acc, y_sh, sem0, sem1):
        cid = lax.axis_index("c")
        sid = lax.axis_index("s")
        base = sid * RPT
        ybase = sid * (N // NS)
        pltpu.sync_copy(
            y_hbm.at[pl.ds(ybase, N // NS), pl.ds(cid * D_OUT, D_OUT)],
            y_sh.at[pl.ds(ybase, N // NS)])
        pltpu.sync_copy(z_hbm.at[pl.ds(base, RPT)], acc.at[pl.ds(base, RPT)])
        pltpu.sync_copy(src_hbm.at[sid], src_v)
        pltpu.sync_copy(dst_hbm.at[sid], dst_v)
        pltpu.sync_copy(b_hbm.at[cid], b_v)
        plsc.subcore_barrier()

        pltpu.async_copy(y_sh.at[src_v.at[0]], rows0, sem0)

        @pl.loop(0, KWF, step=2)
        def _(t):
            pltpu.make_async_copy(y_sh.at[src_v.at[t]], rows0, sem0).wait()
            pltpu.async_copy(y_sh.at[src_v.at[t + 1]], rows1, sem1)
            pltpu.sync_copy(rows0, acc.at[dst_v.at[t]], add=True)
            pltpu.make_async_copy(y_sh.at[src_v.at[t + 1]], rows1, sem1).wait()

            @pl.when(t + 2 < KWF)
            def _():
                pltpu.async_copy(y_sh.at[src_v.at[t + 2]], rows0, sem0)

            pltpu.sync_copy(rows1, acc.at[dst_v.at[t + 1]], add=True)

        plsc.subcore_barrier()

        bv0 = b_v[pl.ds(0, 16)]
        bv1 = b_v[pl.ds(16, 16)]

        NR = N % W

        @pl.loop(0, RPT, step=W)
        def _(rb):
            row = base + rb
            pltpu.sync_copy(acc.at[pl.ds(row, W)], acc_v)
            pltpu.sync_copy(y_sh.at[pl.ds(row, W)], y_v)
            pltpu.sync_copy(
                d16_hbm.at[pl.ds(row, W), pl.ds(D_H, 16)], d_v)

            @pl.loop(0, W)
            def _(r):
                dv = d_v[r]
                a0 = acc_v[r, pl.ds(0, 16)]
                y0 = y_v[r, pl.ds(0, 16)]
                acc_v[r, pl.ds(0, 16)] = dv * (a0 + y0) + bv0
                a1 = acc_v[r, pl.ds(16, 16)]
                y1 = y_v[r, pl.ds(16, 16)]
                acc_v[r, pl.ds(16, 16)] = dv * (a1 + y1) + bv1

            @pl.when(jnp.logical_and(row + W <= N, cid == 0))
            def _():
                pltpu.sync_copy(acc_v, mu_hbm.at[pl.ds(row, W)])

            @pl.when(jnp.logical_and(row + W <= N, cid == 1))
            def _():
                pltpu.sync_copy(acc_v, ls_hbm.at[pl.ds(row, W)])

            @pl.when(jnp.logical_and(row < N, row + W > N))
            def _():
                @pl.when(cid == 0)
                def _():
                    pltpu.sync_copy(acc_v.at[pl.ds(0, NR)],
                                    mu_hbm.at[pl.ds(row, NR)])

                @pl.when(cid == 1)
                def _():
                    pltpu.sync_copy(acc_v.at[pl.ds(0, NR)],
                                    ls_hbm.at[pl.ds(row, NR)])

    return k(y2, srcw2, dstw2, zeros32, d16, b2)


def _tc_layer1(cnt, x, W1):
    def body(c_ref, x_ref, w_ref, a_ref):
        c = c_ref[:, 0:1] + c_ref[:, 16:17]
        d_full = lax.rsqrt(c + 1.0)
        u = jax.lax.dot_general(
            x_ref[...], w_ref[...], (((1,), (0,)), ((), ())),
            preferred_element_type=jnp.float32,
            precision=lax.Precision.HIGHEST)
        y1 = u * d_full[:N]
        a_ref[:N, :D_H] = y1
        a_ref[N:, :D_H] = jnp.zeros((NP - N, D_H), jnp.float32)
        a_ref[:, D_H:D_H + 16] = jnp.broadcast_to(d_full, (NP, 16))
        a_ref[:, D_H + 16:] = jnp.zeros((NP, 48), jnp.float32)

    return pl.pallas_call(
        body,
        out_shape=jax.ShapeDtypeStruct((NP, 128), jnp.float32),
    )(cnt, x, W1)


def _tc_layer2(parts1, a1, Wcat, b1):
    def body(p_ref, a_ref, w_ref, b_ref, o_ref):
        s1 = p_ref[:N, :D_H] + p_ref[:N, D_H:]
        d = a_ref[:N, D_H:D_H + 1]
        y1 = a_ref[:N, :D_H]
        h = jnp.maximum(d * (s1 + y1) + b_ref[...], 0.0)
        y2 = jax.lax.dot_general(
            h, w_ref[...], (((1,), (0,)), ((), ())),
            preferred_element_type=jnp.float32,
            precision=lax.Precision.HIGHEST) * d
        o_ref[:N, :D_H] = y2
        o_ref[N:, :D_H] = jnp.zeros((NP - N, D_H), jnp.float32)
        o_ref[:, D_H:] = jnp.zeros((NP, D_H), jnp.float32)

    return pl.pallas_call(
        body,
        out_shape=jax.ShapeDtypeStruct((NP, 128), jnp.float32),
    )(parts1, a1, Wcat, b1)


def kernel(x, edge_index, W1, b1, Wmu, bmu, Wls, bls):
    src = edge_index[0]
    dst = edge_index[1]
    pad = EP - E
    pad_dst = N + (jnp.arange(pad, dtype=jnp.int32) % (NP - N))
    src_p = jnp.concatenate([src, jnp.zeros((pad,), jnp.int32)])
    dst_p = jnp.concatenate([dst, pad_dst])
    srcw = src_p.reshape(NC, NS, KW, W)
    dstw = dst_p.reshape(NC, NS, KW, W)
    srcw2 = src_p.reshape(NS, KWF, W)
    dstw2 = dst_p.reshape(NS, KWF, W)
    zeros1 = jnp.zeros((NP, 16), jnp.float32)
    zeros64 = jnp.zeros((NP, D_H), jnp.float32)
    zeros32 = jnp.zeros((NP, D_OUT), jnp.float32)
    ones1 = jnp.zeros((W, 16), jnp.float32).at[:, 0].set(1.0)

    cnt = _sc_degree(dstw, ones1, zeros1)
    a1 = _tc_layer1(cnt, x, W1)
    parts1 = _sc_segment_sum(a1, srcw, dstw, zeros64)
    Wcat = jnp.concatenate([Wmu, Wls], axis=1)
    y2 = _tc_layer2(parts1, a1, Wcat, b1.reshape(1, D_H))
    mu, logstd = _sc_segment_out(y2, srcw2, dstw2, zeros32, a1,
                                 jnp.stack([bmu, bls]))
    return (mu, logstd)
